# baseline (speedup 1.0000x reference)
import functools

import jax
import jax.numpy as jnp
from jax.experimental import pallas as pl
from jax.experimental.pallas import tpu as pltpu

STATE_DIM = 4096
HIDDEN = 1024
MAX_PARALLEL = 32
SEQ_LEN = 2048
BATCH = 128

_NEG_INF = float("-inf")


def _layernorm(x, g, b, eps=1e-5):
    mu = jnp.mean(x, axis=-1, keepdims=True)
    xc = x - mu
    var = jnp.mean(xc * xc, axis=-1, keepdims=True)
    return xc * jax.lax.rsqrt(var + eps) * g + b


def _dot_nt(a, b):
    return jax.lax.dot_general(
        a, b, (((1,), (1,)), ((), ())), preferred_element_type=jnp.float32
    )


def _fused_kernel(state_ref, maskf_ref,
                  W1_ref, b1_ref, g1_ref, be1_ref,
                  W2_ref, b2_ref, g2_ref, be2_ref,
                  Wp1_ref, bp1_ref, Wp2_ref, bp2_ref,
                  Wq1_ref, bq1_ref, Wq2_ref, bq2_ref,
                  par_ref, pos_ref):
    state = state_ref[...]
    h = jnp.maximum(_dot_nt(state, W1_ref[...]) + b1_ref[...], 0.0)
    h = _layernorm(h, g1_ref[...], be1_ref[...])
    h = jnp.maximum(_dot_nt(h, W2_ref[...]) + b2_ref[...], 0.0)
    features = _layernorm(h, g2_ref[...], be2_ref[...])

    ph = jnp.maximum(_dot_nt(features, Wp1_ref[...]) + bp1_ref[...], 0.0)
    par = _dot_nt(ph, Wp2_ref[...]) + bp2_ref[...]
    maskf = maskf_ref[...]
    remaining = (SEQ_LEN - jnp.sum(maskf, axis=-1, keepdims=True)).astype(jnp.int32)
    col = jax.lax.broadcasted_iota(jnp.int32, (BATCH, MAX_PARALLEL), 1)
    par_ref[...] = jnp.where(col >= remaining, _NEG_INF, par)

    qh = jnp.maximum(_dot_nt(features, Wq1_ref[...]) + bq1_ref[...], 0.0)
    pos = _dot_nt(qh, Wq2_ref[...]) + bq2_ref[...]
    pos_ref[...] = jnp.where(maskf > 0.0, _NEG_INF, pos)


@jax.jit
def kernel(state, generated_mask, W1, b1, g1, be1, W2, b2, g2, be2,
           Wp1, bp1, Wp2, bp2, Wq1, bq1, Wq2, bq2):
    maskf = generated_mask.astype(jnp.float32)
    vec = lambda v: v.reshape(1, -1)
    full = lambda x: pl.BlockSpec(x.shape, lambda: (0,) * x.ndim)
    args = (state, maskf,
            W1, vec(b1), vec(g1), vec(be1),
            W2, vec(b2), vec(g2), vec(be2),
            Wp1, vec(bp1), Wp2, vec(bp2),
            Wq1, vec(bq1), Wq2, vec(bq2))
    par, pos = pl.pallas_call(
        _fused_kernel,
        grid=(),
        in_specs=[full(a) for a in args],
        out_specs=(
            pl.BlockSpec((BATCH, MAX_PARALLEL), lambda: (0, 0)),
            pl.BlockSpec((BATCH, SEQ_LEN), lambda: (0, 0)),
        ),
        out_shape=(
            jax.ShapeDtypeStruct((BATCH, MAX_PARALLEL), jnp.float32),
            jax.ShapeDtypeStruct((BATCH, SEQ_LEN), jnp.float32),
        ),
        compiler_params=pltpu.CompilerParams(
            vmem_limit_bytes=100 * 1024 * 1024,
        ),
    )(*args)
    return (par, pos)


# speedup vs baseline: 1.1771x; 1.1771x over previous
import jax
import jax.numpy as jnp
from jax.experimental import pallas as pl
from jax.experimental.pallas import tpu as pltpu

STATE_DIM = 4096
HIDDEN = 1024
MAX_PARALLEL = 32
SEQ_LEN = 2048
BATCH = 128

_NEG_INF = float("-inf")
_W1_CHUNKS = 4
_W1_ROWS = HIDDEN // _W1_CHUNKS


def _layernorm(x, g, b, eps=1e-5):
    mu = jnp.mean(x, axis=-1, keepdims=True)
    xc = x - mu
    var = jnp.mean(xc * xc, axis=-1, keepdims=True)
    return xc * jax.lax.rsqrt(var + eps) * g + b


def _dot_nt(a, b):
    return jax.lax.dot_general(
        a, b, (((1,), (1,)), ((), ())), preferred_element_type=jnp.float32
    )


def _fused_kernel(state_ref, maskf_ref,
                  b1_ref, g1_ref, be1_ref,
                  b2_ref, g2_ref, be2_ref,
                  bp1_ref, Wp2_ref, bp2_ref,
                  bq1_ref, bq2_ref,
                  W1_hbm, W2_hbm, Wp1_hbm, Wq1_hbm, Wq2_hbm,
                  par_ref, pos_ref,
                  w1_buf, w2_buf, wp1_buf, wq1_buf, wq2_buf, h_buf,
                  sems):
    copies = [
        pltpu.make_async_copy(
            W1_hbm.at[pl.ds(i * _W1_ROWS, _W1_ROWS), :], w1_buf.at[i],
            sems.at[i])
        for i in range(_W1_CHUNKS)
    ]
    copies.append(pltpu.make_async_copy(W2_hbm, w2_buf, sems.at[_W1_CHUNKS]))
    copies.append(
        pltpu.make_async_copy(Wq1_hbm, wq1_buf, sems.at[_W1_CHUNKS + 1]))
    copies.append(
        pltpu.make_async_copy(Wp1_hbm, wp1_buf, sems.at[_W1_CHUNKS + 2]))
    copies.append(
        pltpu.make_async_copy(Wq2_hbm, wq2_buf, sems.at[_W1_CHUNKS + 3]))
    for c in copies:
        c.start()

    state = state_ref[...]
    for i in range(_W1_CHUNKS):
        copies[i].wait()
        h_buf[:, i * _W1_ROWS:(i + 1) * _W1_ROWS] = _dot_nt(state, w1_buf[i])

    h = jnp.maximum(h_buf[...] + b1_ref[...], 0.0)
    h = _layernorm(h, g1_ref[...], be1_ref[...])

    copies[_W1_CHUNKS].wait()
    h = jnp.maximum(_dot_nt(h, w2_buf[...]) + b2_ref[...], 0.0)
    features = _layernorm(h, g2_ref[...], be2_ref[...])

    copies[_W1_CHUNKS + 1].wait()
    qh = jnp.maximum(_dot_nt(features, wq1_buf[...]) + bq1_ref[...], 0.0)

    copies[_W1_CHUNKS + 2].wait()
    ph = jnp.maximum(_dot_nt(features, wp1_buf[...]) + bp1_ref[...], 0.0)
    par = _dot_nt(ph, Wp2_ref[...]) + bp2_ref[...]
    maskf = maskf_ref[...]
    remaining = (SEQ_LEN - jnp.sum(maskf, axis=-1, keepdims=True)).astype(jnp.int32)
    col = jax.lax.broadcasted_iota(jnp.int32, (BATCH, MAX_PARALLEL), 1)
    par_ref[...] = jnp.where(col >= remaining, _NEG_INF, par)

    copies[_W1_CHUNKS + 3].wait()
    pos = _dot_nt(qh, wq2_buf[...]) + bq2_ref[...]
    pos_ref[...] = jnp.where(maskf > 0.0, _NEG_INF, pos)


@jax.jit
def kernel(state, generated_mask, W1, b1, g1, be1, W2, b2, g2, be2,
           Wp1, bp1, Wp2, bp2, Wq1, bq1, Wq2, bq2):
    maskf = generated_mask.astype(jnp.float32)
    vec = lambda v: v.reshape(1, -1)
    vmem = lambda x: pl.BlockSpec(x.shape, lambda: (0,) * x.ndim)
    hbm = pl.BlockSpec(memory_space=pl.ANY)
    vmem_args = (state, maskf,
                 vec(b1), vec(g1), vec(be1),
                 vec(b2), vec(g2), vec(be2),
                 vec(bp1), Wp2, vec(bp2),
                 vec(bq1), vec(bq2))
    hbm_args = (W1, W2, Wp1, Wq1, Wq2)
    par, pos = pl.pallas_call(
        _fused_kernel,
        grid=(),
        in_specs=[vmem(a) for a in vmem_args] + [hbm] * len(hbm_args),
        out_specs=(
            pl.BlockSpec((BATCH, MAX_PARALLEL), lambda: (0, 0)),
            pl.BlockSpec((BATCH, SEQ_LEN), lambda: (0, 0)),
        ),
        out_shape=(
            jax.ShapeDtypeStruct((BATCH, MAX_PARALLEL), jnp.float32),
            jax.ShapeDtypeStruct((BATCH, SEQ_LEN), jnp.float32),
        ),
        scratch_shapes=[
            pltpu.VMEM((_W1_CHUNKS, _W1_ROWS, STATE_DIM), jnp.float32),
            pltpu.VMEM((HIDDEN, HIDDEN), jnp.float32),
            pltpu.VMEM((HIDDEN // 2, HIDDEN), jnp.float32),
            pltpu.VMEM((HIDDEN, HIDDEN), jnp.float32),
            pltpu.VMEM((SEQ_LEN, HIDDEN), jnp.float32),
            pltpu.VMEM((BATCH, HIDDEN), jnp.float32),
            pltpu.SemaphoreType.DMA((_W1_CHUNKS + 4,)),
        ],
        compiler_params=pltpu.CompilerParams(
            vmem_limit_bytes=100 * 1024 * 1024,
        ),
    )(*vmem_args, *hbm_args)
    return (par, pos)


# speedup vs baseline: 1.2524x; 1.0640x over previous
import jax
import jax.numpy as jnp
from jax.experimental import pallas as pl
from jax.experimental.pallas import tpu as pltpu

STATE_DIM = 4096
HIDDEN = 1024
MAX_PARALLEL = 32
SEQ_LEN = 2048
BATCH = 128

_NEG_INF = float("-inf")
_N1 = 4
_N2 = 2
_NQ1 = 2
_NQ2 = 4


def _layernorm(x, g, b, eps=1e-5):
    mu = jnp.mean(x, axis=-1, keepdims=True)
    xc = x - mu
    var = jnp.mean(xc * xc, axis=-1, keepdims=True)
    return xc * jax.lax.rsqrt(var + eps) * g + b


def _dot_nt(a, b):
    return jax.lax.dot_general(
        a, b, (((1,), (1,)), ((), ())), preferred_element_type=jnp.float32
    )


def _fused_kernel(b1_ref, g1_ref, be1_ref,
                  b2_ref, g2_ref, be2_ref,
                  bp1_ref, Wp2_ref, bp2_ref,
                  bq1_ref, bq2_ref,
                  state_hbm, mask_hbm,
                  W1_hbm, W2_hbm, Wp1_hbm, Wq1_hbm, Wq2_hbm,
                  par_ref, pos_ref,
                  st_buf, mask_buf, w1_buf, w2_buf, wp1_buf, wq1_buf, wq2_buf,
                  h_buf, sems):
    copies = []

    def enqueue(src, dst):
        c = pltpu.make_async_copy(src, dst, sems.at[len(copies)])
        copies.append(c)
        return c

    c_state = enqueue(state_hbm, st_buf)
    c_w1 = [enqueue(W1_hbm.at[pl.ds(i * (HIDDEN // _N1), HIDDEN // _N1), :],
                    w1_buf.at[i]) for i in range(_N1)]
    c_w2 = [enqueue(W2_hbm.at[pl.ds(i * (HIDDEN // _N2), HIDDEN // _N2), :],
                    w2_buf.at[i]) for i in range(_N2)]
    c_mask = enqueue(mask_hbm, mask_buf)
    c_wq1 = [enqueue(Wq1_hbm.at[pl.ds(i * (HIDDEN // _NQ1), HIDDEN // _NQ1), :],
                     wq1_buf.at[i]) for i in range(_NQ1)]
    c_wp1 = enqueue(Wp1_hbm, wp1_buf)
    c_wq2 = [enqueue(Wq2_hbm.at[pl.ds(i * (SEQ_LEN // _NQ2), SEQ_LEN // _NQ2), :],
                     wq2_buf.at[i]) for i in range(_NQ2)]
    for c in copies:
        c.start()

    c_state.wait()
    state = st_buf[...]
    n1 = HIDDEN // _N1
    for i in range(_N1):
        c_w1[i].wait()
        h_buf[:, i * n1:(i + 1) * n1] = _dot_nt(state, w1_buf[i])

    h = jnp.maximum(h_buf[...] + b1_ref[...], 0.0)
    h = _layernorm(h, g1_ref[...], be1_ref[...])

    n2 = HIDDEN // _N2
    parts = []
    for i in range(_N2):
        c_w2[i].wait()
        parts.append(_dot_nt(h, w2_buf[i]))
    h = jnp.maximum(jnp.concatenate(parts, axis=1) + b2_ref[...], 0.0)
    features = _layernorm(h, g2_ref[...], be2_ref[...])

    parts = []
    for i in range(_NQ1):
        c_wq1[i].wait()
        parts.append(_dot_nt(features, wq1_buf[i]))
    qh = jnp.maximum(jnp.concatenate(parts, axis=1) + bq1_ref[...], 0.0)

    c_wp1.wait()
    ph = jnp.maximum(_dot_nt(features, wp1_buf[...]) + bp1_ref[...], 0.0)
    par = _dot_nt(ph, Wp2_ref[...]) + bp2_ref[...]
    c_mask.wait()
    mask = mask_buf[...].astype(jnp.float32)
    remaining = (SEQ_LEN - jnp.sum(mask, axis=-1,
                                   keepdims=True)).astype(jnp.int32)
    col = jax.lax.broadcasted_iota(jnp.int32, (BATCH, MAX_PARALLEL), 1)
    par_ref[...] = jnp.where(col >= remaining, _NEG_INF, par)

    nq2 = SEQ_LEN // _NQ2
    for i in range(_NQ2):
        c_wq2[i].wait()
        sl = slice(i * nq2, (i + 1) * nq2)
        pos = _dot_nt(qh, wq2_buf[i]) + bq2_ref[:, sl]
        pos_ref[:, sl] = jnp.where(mask[:, sl] > 0, _NEG_INF, pos)


@jax.jit
def kernel(state, generated_mask, W1, b1, g1, be1, W2, b2, g2, be2,
           Wp1, bp1, Wp2, bp2, Wq1, bq1, Wq2, bq2):
    mask8 = generated_mask.astype(jnp.int8)
    vec = lambda v: v.reshape(1, -1)
    vmem = lambda x: pl.BlockSpec(x.shape, lambda: (0,) * x.ndim)
    hbm = pl.BlockSpec(memory_space=pl.ANY)
    vmem_args = (vec(b1), vec(g1), vec(be1),
                 vec(b2), vec(g2), vec(be2),
                 vec(bp1), Wp2, vec(bp2),
                 vec(bq1), vec(bq2))
    hbm_args = (state, mask8, W1, W2, Wp1, Wq1, Wq2)
    par, pos = pl.pallas_call(
        _fused_kernel,
        grid=(),
        in_specs=[vmem(a) for a in vmem_args] + [hbm] * len(hbm_args),
        out_specs=(
            pl.BlockSpec((BATCH, MAX_PARALLEL), lambda: (0, 0)),
            pl.BlockSpec((BATCH, SEQ_LEN), lambda: (0, 0)),
        ),
        out_shape=(
            jax.ShapeDtypeStruct((BATCH, MAX_PARALLEL), jnp.float32),
            jax.ShapeDtypeStruct((BATCH, SEQ_LEN), jnp.float32),
        ),
        scratch_shapes=[
            pltpu.VMEM((BATCH, STATE_DIM), jnp.float32),
            pltpu.VMEM((BATCH, SEQ_LEN), jnp.int8),
            pltpu.VMEM((_N1, HIDDEN // _N1, STATE_DIM), jnp.float32),
            pltpu.VMEM((_N2, HIDDEN // _N2, HIDDEN), jnp.float32),
            pltpu.VMEM((HIDDEN // 2, HIDDEN), jnp.float32),
            pltpu.VMEM((_NQ1, HIDDEN // _NQ1, HIDDEN), jnp.float32),
            pltpu.VMEM((_NQ2, SEQ_LEN // _NQ2, HIDDEN), jnp.float32),
            pltpu.VMEM((BATCH, HIDDEN), jnp.float32),
            pltpu.SemaphoreType.DMA((_N1 + _N2 + _NQ1 + _NQ2 + 4,)),
        ],
        compiler_params=pltpu.CompilerParams(
            vmem_limit_bytes=100 * 1024 * 1024,
        ),
    )(*vmem_args, *hbm_args)
    return (par, pos)


# speedup vs baseline: 1.2591x; 1.0053x over previous
import jax
import jax.numpy as jnp
from jax.experimental import pallas as pl
from jax.experimental.pallas import tpu as pltpu

STATE_DIM = 4096
HIDDEN = 1024
MAX_PARALLEL = 32
SEQ_LEN = 2048
BATCH = 128

_NEG_INF = float("-inf")
_N1 = 8
_N2 = 2
_NQ1 = 2
_NQ2 = 4
_LOOKAHEAD = 3


def _layernorm(x, g, b, eps=1e-5):
    mu = jnp.mean(x, axis=-1, keepdims=True)
    xc = x - mu
    var = jnp.mean(xc * xc, axis=-1, keepdims=True)
    return xc * jax.lax.rsqrt(var + eps) * g + b


def _dot_nt(a, b):
    return jax.lax.dot_general(
        a, b, (((1,), (1,)), ((), ())), preferred_element_type=jnp.float32
    )


def _fused_kernel(b1_ref, g1_ref, be1_ref,
                  b2_ref, g2_ref, be2_ref,
                  bp1_ref, Wp2_ref, bp2_ref,
                  bq1_ref, bq2_ref,
                  state_hbm, mask_hbm,
                  W1_hbm, W2_hbm, Wp1_hbm, Wq1_hbm, Wq2_hbm,
                  pos_hbm,
                  par_ref,
                  st_buf, mask_buf, w1_buf, w2_buf, wp1_buf, wq1_buf, wq2_buf,
                  h_buf, pos_buf, sems, out_sems):
    copies = []

    def enqueue(src, dst):
        copies.append(pltpu.make_async_copy(src, dst, sems.at[len(copies)]))
        return len(copies) - 1

    def chunks(hbm_ref, buf, n):
        rows = hbm_ref.shape[0] // n
        return [enqueue(hbm_ref.at[pl.ds(i * rows, rows), :], buf.at[i])
                for i in range(n)]

    i_state = enqueue(state_hbm, st_buf)
    i_w1 = chunks(W1_hbm, w1_buf, _N1)
    i_w2 = chunks(W2_hbm, w2_buf, _N2)
    i_mask = enqueue(mask_hbm, mask_buf)
    i_wq1 = chunks(Wq1_hbm, wq1_buf, _NQ1)
    i_wp1 = enqueue(Wp1_hbm, wp1_buf)
    i_wq2 = chunks(Wq2_hbm, wq2_buf, _NQ2)

    started = [0]

    def wait(idx):
        upto = min(idx + 1 + _LOOKAHEAD, len(copies))
        while started[0] < upto:
            copies[started[0]].start()
            started[0] += 1
        copies[idx].wait()

    wait(i_state)
    state = st_buf[...]
    n1 = HIDDEN // _N1
    for k, idx in enumerate(i_w1):
        wait(idx)
        h_buf[:, k * n1:(k + 1) * n1] = _dot_nt(state, w1_buf[k])

    h = jnp.maximum(h_buf[...] + b1_ref[...], 0.0)
    h = _layernorm(h, g1_ref[...], be1_ref[...])

    parts = []
    for k, idx in enumerate(i_w2):
        wait(idx)
        parts.append(_dot_nt(h, w2_buf[k]))
    h = jnp.maximum(jnp.concatenate(parts, axis=1) + b2_ref[...], 0.0)
    features = _layernorm(h, g2_ref[...], be2_ref[...])

    wait(i_mask)
    mask = mask_buf[...].astype(jnp.float32)

    parts = []
    for k, idx in enumerate(i_wq1):
        wait(idx)
        parts.append(_dot_nt(features, wq1_buf[k]))
    qh = jnp.maximum(jnp.concatenate(parts, axis=1) + bq1_ref[...], 0.0)

    wait(i_wp1)
    ph = jnp.maximum(_dot_nt(features, wp1_buf[...]) + bp1_ref[...], 0.0)
    par = _dot_nt(ph, Wp2_ref[...]) + bp2_ref[...]
    remaining = (SEQ_LEN - jnp.sum(mask, axis=-1,
                                   keepdims=True)).astype(jnp.int32)
    col = jax.lax.broadcasted_iota(jnp.int32, (BATCH, MAX_PARALLEL), 1)
    par_ref[...] = jnp.where(col >= remaining, _NEG_INF, par)

    nq2 = SEQ_LEN // _NQ2
    out_copies = []
    for k, idx in enumerate(i_wq2):
        wait(idx)
        sl = slice(k * nq2, (k + 1) * nq2)
        pos = _dot_nt(qh, wq2_buf[k]) + bq2_ref[:, sl]
        pos_buf[:, sl] = jnp.where(mask[:, sl] > 0, _NEG_INF, pos)
        oc = pltpu.make_async_copy(
            pos_buf.at[:, pl.ds(k * nq2, nq2)],
            pos_hbm.at[:, pl.ds(k * nq2, nq2)],
            out_sems.at[k])
        oc.start()
        out_copies.append(oc)
    for oc in out_copies:
        oc.wait()


@jax.jit
def kernel(state, generated_mask, W1, b1, g1, be1, W2, b2, g2, be2,
           Wp1, bp1, Wp2, bp2, Wq1, bq1, Wq2, bq2):
    mask8 = generated_mask.astype(jnp.int8)
    vec = lambda v: v.reshape(1, -1)
    vmem = lambda x: pl.BlockSpec(x.shape, lambda: (0,) * x.ndim)
    hbm = pl.BlockSpec(memory_space=pl.ANY)
    vmem_args = (vec(b1), vec(g1), vec(be1),
                 vec(b2), vec(g2), vec(be2),
                 vec(bp1), Wp2, vec(bp2),
                 vec(bq1), vec(bq2))
    hbm_args = (state, mask8, W1, W2, Wp1, Wq1, Wq2)
    pos, par = pl.pallas_call(
        _fused_kernel,
        grid=(),
        in_specs=[vmem(a) for a in vmem_args] + [hbm] * len(hbm_args),
        out_specs=(
            pl.BlockSpec(memory_space=pl.ANY),
            pl.BlockSpec((BATCH, MAX_PARALLEL), lambda: (0, 0)),
        ),
        out_shape=(
            jax.ShapeDtypeStruct((BATCH, SEQ_LEN), jnp.float32),
            jax.ShapeDtypeStruct((BATCH, MAX_PARALLEL), jnp.float32),
        ),
        scratch_shapes=[
            pltpu.VMEM((BATCH, STATE_DIM), jnp.float32),
            pltpu.VMEM((BATCH, SEQ_LEN), jnp.int8),
            pltpu.VMEM((_N1, HIDDEN // _N1, STATE_DIM), jnp.float32),
            pltpu.VMEM((_N2, HIDDEN // _N2, HIDDEN), jnp.float32),
            pltpu.VMEM((HIDDEN // 2, HIDDEN), jnp.float32),
            pltpu.VMEM((_NQ1, HIDDEN // _NQ1, HIDDEN), jnp.float32),
            pltpu.VMEM((_NQ2, SEQ_LEN // _NQ2, HIDDEN), jnp.float32),
            pltpu.VMEM((BATCH, HIDDEN), jnp.float32),
            pltpu.VMEM((BATCH, SEQ_LEN), jnp.float32),
            pltpu.SemaphoreType.DMA((_N1 + _N2 + _NQ1 + _NQ2 + 4,)),
            pltpu.SemaphoreType.DMA((_NQ2,)),
        ],
        compiler_params=pltpu.CompilerParams(
            vmem_limit_bytes=100 * 1024 * 1024,
        ),
    )(*vmem_args, *hbm_args)
    return (par, pos)


# speedup vs baseline: 1.2699x; 1.0086x over previous
import jax
import jax.numpy as jnp
from jax.experimental import pallas as pl
from jax.experimental.pallas import tpu as pltpu

STATE_DIM = 4096
HIDDEN = 1024
MAX_PARALLEL = 32
SEQ_LEN = 2048
BATCH = 128

_NEG_INF = float("-inf")
_N1 = 8
_N2 = 2
_NQ1 = 2
_NQ2 = 4
_LOOKAHEAD = 3


def _layernorm(x, g, b, eps=1e-5):
    mu = jnp.mean(x, axis=-1, keepdims=True)
    xc = x - mu
    var = jnp.mean(xc * xc, axis=-1, keepdims=True)
    return xc * jax.lax.rsqrt(var + eps) * g + b


def _dot_nt(a, b):
    return jax.lax.dot_general(
        a.astype(jnp.bfloat16), b.astype(jnp.bfloat16),
        (((1,), (1,)), ((), ())), preferred_element_type=jnp.float32
    )


def _fused_kernel(b1_ref, g1_ref, be1_ref,
                  b2_ref, g2_ref, be2_ref,
                  bp1_ref, Wp2_ref, bp2_ref,
                  bq1_ref, bq2_ref,
                  state_hbm, mask_hbm,
                  W1_hbm, W2_hbm, Wp1_hbm, Wq1_hbm, Wq2_hbm,
                  pos_hbm,
                  par_ref,
                  st_buf, mask_buf, w1_buf, w2_buf, wp1_buf, wq1_buf, wq2_buf,
                  h_buf, pos_buf, sems, out_sems):
    copies = []

    def enqueue(src, dst):
        copies.append(pltpu.make_async_copy(src, dst, sems.at[len(copies)]))
        return len(copies) - 1

    def chunks(hbm_ref, buf, n):
        rows = hbm_ref.shape[0] // n
        return [enqueue(hbm_ref.at[pl.ds(i * rows, rows), :], buf.at[i])
                for i in range(n)]

    i_state = enqueue(state_hbm, st_buf)
    i_w1 = chunks(W1_hbm, w1_buf, _N1)
    i_w2 = chunks(W2_hbm, w2_buf, _N2)
    i_mask = enqueue(mask_hbm, mask_buf)
    i_wq1 = chunks(Wq1_hbm, wq1_buf, _NQ1)
    i_wp1 = enqueue(Wp1_hbm, wp1_buf)
    i_wq2 = chunks(Wq2_hbm, wq2_buf, _NQ2)

    started = [0]

    def wait(idx):
        upto = min(idx + 1 + _LOOKAHEAD, len(copies))
        while started[0] < upto:
            copies[started[0]].start()
            started[0] += 1
        copies[idx].wait()

    wait(i_state)
    state = st_buf[...]
    n1 = HIDDEN // _N1
    for k, idx in enumerate(i_w1):
        wait(idx)
        h_buf[:, k * n1:(k + 1) * n1] = _dot_nt(state, w1_buf[k])

    h = jnp.maximum(h_buf[...] + b1_ref[...], 0.0)
    h = _layernorm(h, g1_ref[...], be1_ref[...])

    parts = []
    for k, idx in enumerate(i_w2):
        wait(idx)
        parts.append(_dot_nt(h, w2_buf[k]))
    h = jnp.maximum(jnp.concatenate(parts, axis=1) + b2_ref[...], 0.0)
    features = _layernorm(h, g2_ref[...], be2_ref[...])

    wait(i_mask)
    mask = mask_buf[...].astype(jnp.float32)

    parts = []
    for k, idx in enumerate(i_wq1):
        wait(idx)
        parts.append(_dot_nt(features, wq1_buf[k]))
    qh = jnp.maximum(jnp.concatenate(parts, axis=1) + bq1_ref[...], 0.0)

    wait(i_wp1)
    ph = jnp.maximum(_dot_nt(features, wp1_buf[...]) + bp1_ref[...], 0.0)
    par = _dot_nt(ph, Wp2_ref[...]) + bp2_ref[...]
    remaining = (SEQ_LEN - jnp.sum(mask, axis=-1,
                                   keepdims=True)).astype(jnp.int32)
    col = jax.lax.broadcasted_iota(jnp.int32, (BATCH, MAX_PARALLEL), 1)
    par_ref[...] = jnp.where(col >= remaining, _NEG_INF, par)

    nq2 = SEQ_LEN // _NQ2
    out_copies = []
    for k, idx in enumerate(i_wq2):
        wait(idx)
        sl = slice(k * nq2, (k + 1) * nq2)
        pos = _dot_nt(qh, wq2_buf[k]) + bq2_ref[:, sl]
        pos_buf[:, sl] = jnp.where(mask[:, sl] > 0, _NEG_INF, pos)
        oc = pltpu.make_async_copy(
            pos_buf.at[:, pl.ds(k * nq2, nq2)],
            pos_hbm.at[:, pl.ds(k * nq2, nq2)],
            out_sems.at[k])
        oc.start()
        out_copies.append(oc)
    for oc in out_copies:
        oc.wait()


@jax.jit
def kernel(state, generated_mask, W1, b1, g1, be1, W2, b2, g2, be2,
           Wp1, bp1, Wp2, bp2, Wq1, bq1, Wq2, bq2):
    mask8 = generated_mask.astype(jnp.int8)
    vec = lambda v: v.reshape(1, -1)
    vmem = lambda x: pl.BlockSpec(x.shape, lambda: (0,) * x.ndim)
    hbm = pl.BlockSpec(memory_space=pl.ANY)
    vmem_args = (vec(b1), vec(g1), vec(be1),
                 vec(b2), vec(g2), vec(be2),
                 vec(bp1), Wp2, vec(bp2),
                 vec(bq1), vec(bq2))
    hbm_args = (state, mask8, W1, W2, Wp1, Wq1, Wq2)
    pos, par = pl.pallas_call(
        _fused_kernel,
        grid=(),
        in_specs=[vmem(a) for a in vmem_args] + [hbm] * len(hbm_args),
        out_specs=(
            pl.BlockSpec(memory_space=pl.ANY),
            pl.BlockSpec((BATCH, MAX_PARALLEL), lambda: (0, 0)),
        ),
        out_shape=(
            jax.ShapeDtypeStruct((BATCH, SEQ_LEN), jnp.float32),
            jax.ShapeDtypeStruct((BATCH, MAX_PARALLEL), jnp.float32),
        ),
        scratch_shapes=[
            pltpu.VMEM((BATCH, STATE_DIM), jnp.float32),
            pltpu.VMEM((BATCH, SEQ_LEN), jnp.int8),
            pltpu.VMEM((_N1, HIDDEN // _N1, STATE_DIM), jnp.float32),
            pltpu.VMEM((_N2, HIDDEN // _N2, HIDDEN), jnp.float32),
            pltpu.VMEM((HIDDEN // 2, HIDDEN), jnp.float32),
            pltpu.VMEM((_NQ1, HIDDEN // _NQ1, HIDDEN), jnp.float32),
            pltpu.VMEM((_NQ2, SEQ_LEN // _NQ2, HIDDEN), jnp.float32),
            pltpu.VMEM((BATCH, HIDDEN), jnp.float32),
            pltpu.VMEM((BATCH, SEQ_LEN), jnp.float32),
            pltpu.SemaphoreType.DMA((_N1 + _N2 + _NQ1 + _NQ2 + 4,)),
            pltpu.SemaphoreType.DMA((_NQ2,)),
        ],
        compiler_params=pltpu.CompilerParams(
            vmem_limit_bytes=100 * 1024 * 1024,
        ),
    )(*vmem_args, *hbm_args)
    return (par, pos)
